# baseline (speedup 1.0000x reference)
import functools

import jax
import jax.numpy as jnp
from jax.experimental import pallas as pl
from jax.experimental.pallas import tpu as pltpu
from jax.experimental.pallas import tpu_sc as plsc

PH = 16
PW = 16
HALF = 512

_mesh = plsc.ScalarSubcoreMesh(axis_name="c", num_cores=1)


@functools.partial(
    pl.kernel,
    out_type=jax.ShapeDtypeStruct((PH, PW, 2, HALF), jnp.float32),
    mesh=_mesh,
    scratch_types=[
        pltpu.SemaphoreType.DMA,
    ],
)
def _sc_fill(row_hbm, col_hbm, out_hbm, sem_out):
    descs = []
    for i in range(PH):
        descs.append(pltpu.async_copy(row_hbm, out_hbm.at[:, i, 0], sem_out))
        descs.append(pltpu.async_copy(col_hbm, out_hbm.at[i, :, 1], sem_out))
    for d in descs:
        d.wait()


def kernel(row_embed, col_embed):
    out = _sc_fill(row_embed, col_embed)
    return out.reshape(PH * PW, 2 * HALF)


# speedup vs baseline: 2.4642x; 2.4642x over previous
import functools

import jax
import jax.numpy as jnp
from jax.experimental import pallas as pl
from jax.experimental.pallas import tpu as pltpu
from jax.experimental.pallas import tpu_sc as plsc

PH = 16
PW = 16
HALF = 512

_mesh = plsc.ScalarSubcoreMesh(axis_name="c", num_cores=1)


@functools.partial(
    pl.kernel,
    out_type=jax.ShapeDtypeStruct((PH, PW, 2, HALF), jnp.float32),
    mesh=_mesh,
    scratch_types=[
        pltpu.VMEM_SHARED((PH, HALF), jnp.float32),
        pltpu.VMEM_SHARED((PW, HALF), jnp.float32),
        pltpu.SemaphoreType.DMA,
        pltpu.SemaphoreType.DMA,
        pltpu.SemaphoreType.DMA,
    ],
)
def _sc_fill(row_hbm, col_hbm, out_hbm, rtab, ctab, sem_r, sem_c, sem_out):
    dr = pltpu.async_copy(row_hbm, rtab, sem_r)
    dc = pltpu.async_copy(col_hbm, ctab, sem_c)
    descs = []
    dc.wait()
    for i in range(PH):
        descs.append(pltpu.async_copy(ctab, out_hbm.at[i, :, 1], sem_out))
    dr.wait()
    for i in range(PW):
        descs.append(pltpu.async_copy(rtab, out_hbm.at[:, i, 0], sem_out))
    for d in descs:
        d.wait()


def kernel(row_embed, col_embed):
    out = _sc_fill(row_embed, col_embed)
    return out.reshape(PH * PW, 2 * HALF)
